# Initial kernel scaffold; baseline (speedup 1.0000x reference)
import jax
import jax.numpy as jnp
from jax.experimental import pallas as pl


def kernel(xyz, points, W0, b0, g0, bt0, W1, b1, g1, bt1, W2, b2, g2, bt2):
    raise NotImplementedError("write your pallas kernel here")


# speedup vs baseline: 8.8890x; 8.8890x over previous
import functools

import jax
import jax.numpy as jnp
from jax import lax
from jax.experimental import pallas as pl
from jax.experimental.pallas import tpu as pltpu
from jax.experimental.pallas import tpu_sc as plsc

B = 8
N = 16384
S = 512
K = 32
R2 = 0.2 ** 2
NR = 8
NC_ = N // NR
CHUNK = 2048
NCHUNKS = N // CHUNK
BIGF = 1e10
BIGI = 1 << 30


def _fps_body(xr_ref, nx_ref, ny_ref, nz_ref, dist_ref):
    x = xr_ref[0, 0]
    y = xr_ref[0, 1]
    z = xr_ref[0, 2]
    li = (lax.broadcasted_iota(jnp.int32, (NR, NC_), 0) * NC_
          + lax.broadcasted_iota(jnp.int32, (NR, NC_), 1))
    dist_ref[...] = jnp.full((NR, NC_), BIGF, jnp.float32)

    def step(i, far):
        fmask = li == far
        cx = jnp.sum(jnp.where(fmask, x, 0.0))
        cy = jnp.sum(jnp.where(fmask, y, 0.0))
        cz = jnp.sum(jnp.where(fmask, z, 0.0))
        nx_ref[0, 0, i] = cx
        ny_ref[0, 0, i] = cy
        nz_ref[0, 0, i] = cz
        dx = x - cx
        dy = y - cy
        dz = z - cz
        d = (dx * dx + dy * dy) + dz * dz
        dist = jnp.minimum(dist_ref[...], d)
        dist_ref[...] = dist
        m = jnp.max(dist)
        return jnp.min(jnp.where(dist == m, li, BIGI))

    lax.fori_loop(0, S, step, jnp.int32(0))


def _fps(xyz_r):
    out = [jax.ShapeDtypeStruct((B, 1, S), jnp.float32)] * 3
    spec = pl.BlockSpec((1, 1, S), lambda b: (b, 0, 0), memory_space=pltpu.SMEM)
    return pl.pallas_call(
        _fps_body,
        grid=(B,),
        in_specs=[pl.BlockSpec((1, 3, NR, NC_), lambda b: (b, 0, 0, 0))],
        out_specs=[spec, spec, spec],
        out_shape=out,
        scratch_shapes=[pltpu.VMEM((NR, NC_), jnp.float32)],
    )(xyz_r)


def _ballq_body(xt_ref, nx_ref, ny_ref, nz_ref, out_ref):
    b = pl.program_id(0)
    sx = nx_ref[0]
    sy = ny_ref[0]
    sz = nz_ref[0]
    s2 = (sx * sx + sy * sy) + sz * sz
    src = jnp.concatenate([sx, sy, sz], axis=1)
    kio = lax.broadcasted_iota(jnp.int32, (S, K), 1)

    def cond(carry):
        ci, count, _ = carry
        return jnp.logical_and(ci < NCHUNKS, jnp.min(count) < K)

    def body(carry):
        ci, count, idxout = carry
        off = pl.multiple_of(ci * CHUNK, CHUNK)
        dx = xt_ref[0, 0:1, pl.ds(off, CHUNK)]
        dy = xt_ref[0, 1:2, pl.ds(off, CHUNK)]
        dz = xt_ref[0, 2:3, pl.ds(off, CHUNK)]
        dstc = jnp.concatenate([dx, dy, dz], axis=0)
        dot = lax.dot_general(src, dstc, (((1,), (0,)), ((), ())))
        d2 = (dx * dx + dy * dy) + dz * dz
        d = ((-2.0 * dot) + s2) + d2
        gi = lax.broadcasted_iota(jnp.int32, (S, CHUNK), 1) + off
        mi = jnp.where(d > R2, BIGI, gi)

        def extract(_, c2):
            count2, idxout2, mi2 = c2
            first = jnp.min(mi2, axis=1, keepdims=True)
            take = jnp.logical_and(first < BIGI, count2 < K)
            upd = jnp.logical_and(kio == count2, take)
            idxout2 = jnp.where(upd, first, idxout2)
            mi2 = jnp.where(mi2 == first, BIGI, mi2)
            count2 = count2 + take.astype(jnp.int32)
            return count2, idxout2, mi2

        count, idxout, _ = lax.fori_loop(0, K, extract, (count, idxout, mi))
        return ci + 1, count, idxout

    init = (jnp.int32(0),
            jnp.zeros((S, 1), jnp.int32),
            jnp.zeros((S, K), jnp.int32))
    _, count, idxout = lax.while_loop(cond, body, init)
    idxout = jnp.where(kio >= count, idxout[:, 0:1], idxout)
    out_ref[0] = idxout + b * N


def _ballq(xyz_t, nx, ny, nz):
    nspec = pl.BlockSpec((1, S, 1), lambda b: (b, 0, 0))
    return pl.pallas_call(
        _ballq_body,
        grid=(B,),
        in_specs=[pl.BlockSpec((1, 3, N), lambda b: (b, 0, 0)),
                  nspec, nspec, nspec],
        out_specs=pl.BlockSpec((1, S, K), lambda b: (b, 0, 0)),
        out_shape=jax.ShapeDtypeStruct((B, S, K), jnp.int32),
    )(xyz_t, nx[..., None], ny[..., None], nz[..., None])


ROWS = B * S * K
NW = 32
RPW = ROWS // NW
GC = 128
NCH = RPW // GC
TW = 16


def _sc_gather(table, gidx):
    mesh = plsc.VectorSubcoreMesh(core_axis_name="c", subcore_axis_name="s")

    @functools.partial(
        pl.kernel,
        mesh=mesh,
        compiler_params=pltpu.CompilerParams(use_tc_tiling_on_sc=False),
        out_type=jax.ShapeDtypeStruct((ROWS, TW), jnp.float32),
        scratch_types=[
            pltpu.VMEM((NCH, GC), jnp.int32),
            pltpu.VMEM((GC, TW), jnp.float32),
            pltpu.SemaphoreType.DMA,
        ],
    )
    def k(table_hbm, idx_hbm, out_hbm, idx_v, row_v, sem):
        wid = lax.axis_index("s") * 2 + lax.axis_index("c")
        pltpu.sync_copy(idx_hbm.at[wid], idx_v)
        base = wid * RPW

        def body(c, carry):
            pltpu.async_copy(table_hbm.at[idx_v.at[c]], row_v, sem).wait()
            pltpu.sync_copy(row_v, out_hbm.at[pl.ds(base + c * GC, GC)])
            return carry

        lax.fori_loop(0, NCH, body, jnp.int32(0))

    return k(table, gidx.reshape(NW, NCH, GC))


G = 256
RB = G * K
NBLK = (B * S) // G


def _p1_body(g_ref, nx_ref, w_ref, b_ref, z_ref, s_ref, q_ref, sa, qa):
    i = pl.program_id(0)
    xb = g_ref[...]
    nb = nx_ref[...]
    x0 = (xb - nb[:, None, :]).reshape(RB, TW)
    z = lax.dot_general(x0, w_ref[...], (((1,), (0,)), ((), ()))) + b_ref[...]
    z_ref[...] = z

    @pl.when(i == 0)
    def _():
        sa[...] = jnp.zeros_like(sa)
        qa[...] = jnp.zeros_like(qa)

    sa[...] += jnp.sum(z, axis=0, keepdims=True)
    qa[...] += jnp.sum(z * z, axis=0, keepdims=True)

    @pl.when(i == NBLK - 1)
    def _():
        s_ref[...] = sa[...]
        q_ref[...] = qa[...]


def _p1(gathered, nxpad, w0p, b0):
    co = w0p.shape[1]
    return pl.pallas_call(
        _p1_body,
        grid=(NBLK,),
        in_specs=[
            pl.BlockSpec((G, K, TW), lambda i: (i, 0, 0)),
            pl.BlockSpec((G, TW), lambda i: (i, 0)),
            pl.BlockSpec((TW, co), lambda i: (0, 0)),
            pl.BlockSpec((1, co), lambda i: (0, 0)),
        ],
        out_specs=[
            pl.BlockSpec((RB, co), lambda i: (i, 0)),
            pl.BlockSpec((1, co), lambda i: (0, 0)),
            pl.BlockSpec((1, co), lambda i: (0, 0)),
        ],
        out_shape=[
            jax.ShapeDtypeStruct((ROWS, co), jnp.float32),
            jax.ShapeDtypeStruct((1, co), jnp.float32),
            jax.ShapeDtypeStruct((1, co), jnp.float32),
        ],
        scratch_shapes=[pltpu.VMEM((1, co), jnp.float32),
                        pltpu.VMEM((1, co), jnp.float32)],
    )(gathered.reshape(B * S, K, TW), nxpad, w0p, b0[None, :])


def _pmid_body(z_ref, sc_ref, sh_ref, w_ref, b_ref, z2_ref, s_ref, q_ref,
               sa, qa):
    i = pl.program_id(0)
    y = jnp.maximum(z_ref[...] * sc_ref[...] + sh_ref[...], 0.0)
    z = lax.dot_general(y, w_ref[...], (((1,), (0,)), ((), ()))) + b_ref[...]
    z2_ref[...] = z

    @pl.when(i == 0)
    def _():
        sa[...] = jnp.zeros_like(sa)
        qa[...] = jnp.zeros_like(qa)

    sa[...] += jnp.sum(z, axis=0, keepdims=True)
    qa[...] += jnp.sum(z * z, axis=0, keepdims=True)

    @pl.when(i == NBLK - 1)
    def _():
        s_ref[...] = sa[...]
        q_ref[...] = qa[...]


def _pmid(zin, scale, shift, wp, bv):
    ci = zin.shape[1]
    co = wp.shape[1]
    return pl.pallas_call(
        _pmid_body,
        grid=(NBLK,),
        in_specs=[
            pl.BlockSpec((RB, ci), lambda i: (i, 0)),
            pl.BlockSpec((1, ci), lambda i: (0, 0)),
            pl.BlockSpec((1, ci), lambda i: (0, 0)),
            pl.BlockSpec((ci, co), lambda i: (0, 0)),
            pl.BlockSpec((1, co), lambda i: (0, 0)),
        ],
        out_specs=[
            pl.BlockSpec((RB, co), lambda i: (i, 0)),
            pl.BlockSpec((1, co), lambda i: (0, 0)),
            pl.BlockSpec((1, co), lambda i: (0, 0)),
        ],
        out_shape=[
            jax.ShapeDtypeStruct((ROWS, co), jnp.float32),
            jax.ShapeDtypeStruct((1, co), jnp.float32),
            jax.ShapeDtypeStruct((1, co), jnp.float32),
        ],
        scratch_shapes=[pltpu.VMEM((1, co), jnp.float32),
                        pltpu.VMEM((1, co), jnp.float32)],
    )(zin, scale[None, :], shift[None, :], wp, bv[None, :])


def _p4_body(z_ref, sc_ref, sh_ref, out_ref):
    y = jnp.maximum(z_ref[...] * sc_ref[...][None] + sh_ref[...][None], 0.0)
    out_ref[...] = jnp.max(y, axis=1)


def _p4(z3, scale, shift):
    co = z3.shape[1]
    return pl.pallas_call(
        _p4_body,
        grid=(NBLK,),
        in_specs=[
            pl.BlockSpec((G, K, co), lambda i: (i, 0, 0)),
            pl.BlockSpec((1, co), lambda i: (0, 0)),
            pl.BlockSpec((1, co), lambda i: (0, 0)),
        ],
        out_specs=pl.BlockSpec((G, co), lambda i: (i, 0)),
        out_shape=jax.ShapeDtypeStruct((B * S, co), jnp.float32),
    )(z3.reshape(B * S, K, co), scale[None, :], shift[None, :])


def _stats(s, q, g, bt):
    m = s[0] / ROWS
    v = q[0] / ROWS - m * m
    scale = g * lax.rsqrt(v + 1e-5)
    shift = bt - m * scale
    return scale, shift


def kernel(xyz, points, W0, b0, g0, bt0, W1, b1, g1, bt1, W2, b2, g2, bt2):
    xyz_t = jnp.transpose(xyz, (0, 2, 1))
    xyz_r = xyz_t.reshape(B, 3, NR, NC_)
    nx, ny, nz = _fps(xyz_r)
    nx = nx.reshape(B, S)
    ny = ny.reshape(B, S)
    nz = nz.reshape(B, S)
    gidx = _ballq(xyz_t, nx, ny, nz)

    table = jnp.concatenate(
        [xyz, points, jnp.zeros((B, N, TW - 9), jnp.float32)], axis=-1
    ).reshape(B * N, TW)
    gathered = _sc_gather(table, gidx.reshape(-1))

    new_xyz = jnp.stack([nx, ny, nz], axis=-1)
    nxpad = jnp.concatenate(
        [new_xyz.reshape(B * S, 3), jnp.zeros((B * S, TW - 3), jnp.float32)],
        axis=-1)

    w0p = jnp.zeros((TW, 32), jnp.float32).at[:9, :].set(W0.T)
    z1, s1, q1 = _p1(gathered, nxpad, w0p, b0)
    sc1, sh1 = _stats(s1, q1, g0, bt0)
    z2, s2, q2 = _pmid(z1, sc1, sh1, W1.T, b1)
    sc2, sh2 = _stats(s2, q2, g1, bt1)
    z3, s3, q3 = _pmid(z2, sc2, sh2, W2.T, b2)
    sc3, sh3 = _stats(s3, q3, g2, bt2)
    new_points = _p4(z3, sc3, sh3).reshape(B, S, 64)
    return (new_xyz, new_points)


# speedup vs baseline: 13.5001x; 1.5187x over previous
import functools

import jax
import jax.numpy as jnp
from jax import lax
from jax.experimental import pallas as pl
from jax.experimental.pallas import tpu as pltpu
from jax.experimental.pallas import tpu_sc as plsc

B = 8
N = 16384
S = 512
K = 32
R2 = 0.2 ** 2
NR = 8
NC_ = N // NR
CHUNK = 512
NCHUNKS = N // CHUNK
BIGF = 1e10
BIGI = 1 << 30


def _fps_body(xr_ref, nx_ref, ny_ref, nz_ref, dist_ref):
    x = xr_ref[:, 0]
    y = xr_ref[:, 1]
    z = xr_ref[:, 2]
    li = (lax.broadcasted_iota(jnp.int32, (B, NR, NC_), 1) * NC_
          + lax.broadcasted_iota(jnp.int32, (B, NR, NC_), 2))
    col = lax.broadcasted_iota(jnp.int32, (B, S), 1)
    dist_ref[...] = jnp.full((B, NR, NC_), BIGF, jnp.float32)

    def step(i, far):
        fmask = li == far
        cx = jnp.sum(jnp.where(fmask, x, 0.0), axis=(1, 2), keepdims=True)
        cy = jnp.sum(jnp.where(fmask, y, 0.0), axis=(1, 2), keepdims=True)
        cz = jnp.sum(jnp.where(fmask, z, 0.0), axis=(1, 2), keepdims=True)
        sel = col == i
        nx_ref[...] = jnp.where(sel, cx[:, :, 0], nx_ref[...])
        ny_ref[...] = jnp.where(sel, cy[:, :, 0], ny_ref[...])
        nz_ref[...] = jnp.where(sel, cz[:, :, 0], nz_ref[...])
        dx = x - cx
        dy = y - cy
        dz = z - cz
        d = (dx * dx + dy * dy) + dz * dz
        dist = jnp.minimum(dist_ref[...], d)
        dist_ref[...] = dist
        m = jnp.max(dist, axis=(1, 2), keepdims=True)
        return jnp.min(jnp.where(dist == m, li, BIGI), axis=(1, 2),
                       keepdims=True)

    lax.fori_loop(0, S, step, jnp.zeros((B, 1, 1), jnp.int32))


def _fps(xyz_r):
    out = [jax.ShapeDtypeStruct((B, S), jnp.float32)] * 3
    spec = pl.BlockSpec((B, S), lambda: (0, 0))
    return pl.pallas_call(
        _fps_body,
        in_specs=[pl.BlockSpec((B, 3, NR, NC_), lambda: (0, 0, 0, 0))],
        out_specs=[spec, spec, spec],
        out_shape=out,
        scratch_shapes=[pltpu.VMEM((B, NR, NC_), jnp.float32)],
    )(xyz_r)


def _ballq_body(xt_ref, nx_ref, ny_ref, nz_ref, out_ref):
    b = pl.program_id(0)
    sx = nx_ref[0]
    sy = ny_ref[0]
    sz = nz_ref[0]
    s2 = (sx * sx + sy * sy) + sz * sz
    src = jnp.concatenate([sx, sy, sz], axis=1)
    kio = lax.broadcasted_iota(jnp.int32, (S, K), 1)

    def cond(carry):
        ci, count, _ = carry
        return jnp.logical_and(ci < NCHUNKS, jnp.min(count) < K)

    def body(carry):
        ci, count, idxout = carry
        off = pl.multiple_of(ci * CHUNK, CHUNK)
        dx = xt_ref[0, 0:1, pl.ds(off, CHUNK)]
        dy = xt_ref[0, 1:2, pl.ds(off, CHUNK)]
        dz = xt_ref[0, 2:3, pl.ds(off, CHUNK)]
        dstc = jnp.concatenate([dx, dy, dz], axis=0)
        dot = lax.dot_general(src, dstc, (((1,), (0,)), ((), ())))
        d2 = (dx * dx + dy * dy) + dz * dz
        d = ((-2.0 * dot) + s2) + d2
        gi = lax.broadcasted_iota(jnp.int32, (S, CHUNK), 1) + off
        mi = jnp.where(d > R2, BIGI, gi)

        def extract(_, c2):
            count2, idxout2, mi2 = c2
            first = jnp.min(mi2, axis=1, keepdims=True)
            take = jnp.logical_and(first < BIGI, count2 < K)
            upd = jnp.logical_and(kio == count2, take)
            idxout2 = jnp.where(upd, first, idxout2)
            mi2 = jnp.where(mi2 == first, BIGI, mi2)
            count2 = count2 + take.astype(jnp.int32)
            return count2, idxout2, mi2

        count, idxout, _ = lax.fori_loop(0, K, extract, (count, idxout, mi))
        return ci + 1, count, idxout

    init = (jnp.int32(0),
            jnp.zeros((S, 1), jnp.int32),
            jnp.zeros((S, K), jnp.int32))
    _, count, idxout = lax.while_loop(cond, body, init)
    idxout = jnp.where(kio >= count, idxout[:, 0:1], idxout)
    out_ref[0] = idxout + b * N


def _ballq(xyz_t, nx, ny, nz):
    nspec = pl.BlockSpec((1, S, 1), lambda b: (b, 0, 0))
    return pl.pallas_call(
        _ballq_body,
        grid=(B,),
        in_specs=[pl.BlockSpec((1, 3, N), lambda b: (b, 0, 0)),
                  nspec, nspec, nspec],
        out_specs=pl.BlockSpec((1, S, K), lambda b: (b, 0, 0)),
        out_shape=jax.ShapeDtypeStruct((B, S, K), jnp.int32),
    )(xyz_t, nx[..., None], ny[..., None], nz[..., None])


ROWS = B * S * K
NW = 32
RPW = ROWS // NW
GC = 128
NCH = RPW // GC
TW = 16


def _sc_gather(table, gidx):
    mesh = plsc.VectorSubcoreMesh(core_axis_name="c", subcore_axis_name="s")

    @functools.partial(
        pl.kernel,
        mesh=mesh,
        compiler_params=pltpu.CompilerParams(use_tc_tiling_on_sc=False),
        out_type=jax.ShapeDtypeStruct((ROWS, TW), jnp.float32),
        scratch_types=[
            pltpu.VMEM((NCH, GC), jnp.int32),
            pltpu.VMEM((GC, TW), jnp.float32),
            pltpu.SemaphoreType.DMA,
        ],
    )
    def k(table_hbm, idx_hbm, out_hbm, idx_v, row_v, sem):
        wid = lax.axis_index("s") * 2 + lax.axis_index("c")
        pltpu.sync_copy(idx_hbm.at[wid], idx_v)
        base = wid * RPW

        def body(c, carry):
            pltpu.async_copy(table_hbm.at[idx_v.at[c]], row_v, sem).wait()
            pltpu.sync_copy(row_v, out_hbm.at[pl.ds(base + c * GC, GC)])
            return carry

        lax.fori_loop(0, NCH, body, jnp.int32(0))

    return k(table, gidx.reshape(NW, NCH, GC))


G = 256
RB = G * K
NBLK = (B * S) // G


def _p1_body(g_ref, nx_ref, w_ref, b_ref, z_ref, s_ref, q_ref, sa, qa):
    i = pl.program_id(0)
    xb = g_ref[...]
    nb = nx_ref[...]
    x0 = (xb - nb[:, None, :]).reshape(RB, TW)
    z = lax.dot_general(x0, w_ref[...], (((1,), (0,)), ((), ()))) + b_ref[...]
    z_ref[...] = z

    @pl.when(i == 0)
    def _():
        sa[...] = jnp.zeros_like(sa)
        qa[...] = jnp.zeros_like(qa)

    sa[...] += jnp.sum(z, axis=0, keepdims=True)
    qa[...] += jnp.sum(z * z, axis=0, keepdims=True)

    @pl.when(i == NBLK - 1)
    def _():
        s_ref[...] = sa[...]
        q_ref[...] = qa[...]


def _p1(gathered, nxpad, w0p, b0):
    co = w0p.shape[1]
    return pl.pallas_call(
        _p1_body,
        grid=(NBLK,),
        in_specs=[
            pl.BlockSpec((G, K, TW), lambda i: (i, 0, 0)),
            pl.BlockSpec((G, TW), lambda i: (i, 0)),
            pl.BlockSpec((TW, co), lambda i: (0, 0)),
            pl.BlockSpec((1, co), lambda i: (0, 0)),
        ],
        out_specs=[
            pl.BlockSpec((RB, co), lambda i: (i, 0)),
            pl.BlockSpec((1, co), lambda i: (0, 0)),
            pl.BlockSpec((1, co), lambda i: (0, 0)),
        ],
        out_shape=[
            jax.ShapeDtypeStruct((ROWS, co), jnp.float32),
            jax.ShapeDtypeStruct((1, co), jnp.float32),
            jax.ShapeDtypeStruct((1, co), jnp.float32),
        ],
        scratch_shapes=[pltpu.VMEM((1, co), jnp.float32),
                        pltpu.VMEM((1, co), jnp.float32)],
    )(gathered.reshape(B * S, K, TW), nxpad, w0p, b0[None, :])


def _pmid_body(z_ref, sc_ref, sh_ref, w_ref, b_ref, z2_ref, s_ref, q_ref,
               sa, qa):
    i = pl.program_id(0)
    y = jnp.maximum(z_ref[...] * sc_ref[...] + sh_ref[...], 0.0)
    z = lax.dot_general(y, w_ref[...], (((1,), (0,)), ((), ()))) + b_ref[...]
    z2_ref[...] = z

    @pl.when(i == 0)
    def _():
        sa[...] = jnp.zeros_like(sa)
        qa[...] = jnp.zeros_like(qa)

    sa[...] += jnp.sum(z, axis=0, keepdims=True)
    qa[...] += jnp.sum(z * z, axis=0, keepdims=True)

    @pl.when(i == NBLK - 1)
    def _():
        s_ref[...] = sa[...]
        q_ref[...] = qa[...]


def _pmid(zin, scale, shift, wp, bv):
    ci = zin.shape[1]
    co = wp.shape[1]
    return pl.pallas_call(
        _pmid_body,
        grid=(NBLK,),
        in_specs=[
            pl.BlockSpec((RB, ci), lambda i: (i, 0)),
            pl.BlockSpec((1, ci), lambda i: (0, 0)),
            pl.BlockSpec((1, ci), lambda i: (0, 0)),
            pl.BlockSpec((ci, co), lambda i: (0, 0)),
            pl.BlockSpec((1, co), lambda i: (0, 0)),
        ],
        out_specs=[
            pl.BlockSpec((RB, co), lambda i: (i, 0)),
            pl.BlockSpec((1, co), lambda i: (0, 0)),
            pl.BlockSpec((1, co), lambda i: (0, 0)),
        ],
        out_shape=[
            jax.ShapeDtypeStruct((ROWS, co), jnp.float32),
            jax.ShapeDtypeStruct((1, co), jnp.float32),
            jax.ShapeDtypeStruct((1, co), jnp.float32),
        ],
        scratch_shapes=[pltpu.VMEM((1, co), jnp.float32),
                        pltpu.VMEM((1, co), jnp.float32)],
    )(zin, scale[None, :], shift[None, :], wp, bv[None, :])


def _p4_body(z_ref, sc_ref, sh_ref, out_ref):
    y = jnp.maximum(z_ref[...] * sc_ref[...][None] + sh_ref[...][None], 0.0)
    out_ref[...] = jnp.max(y, axis=1)


def _p4(z3, scale, shift):
    co = z3.shape[1]
    return pl.pallas_call(
        _p4_body,
        grid=(NBLK,),
        in_specs=[
            pl.BlockSpec((G, K, co), lambda i: (i, 0, 0)),
            pl.BlockSpec((1, co), lambda i: (0, 0)),
            pl.BlockSpec((1, co), lambda i: (0, 0)),
        ],
        out_specs=pl.BlockSpec((G, co), lambda i: (i, 0)),
        out_shape=jax.ShapeDtypeStruct((B * S, co), jnp.float32),
    )(z3.reshape(B * S, K, co), scale[None, :], shift[None, :])


def _stats(s, q, g, bt):
    m = s[0] / ROWS
    v = q[0] / ROWS - m * m
    scale = g * lax.rsqrt(v + 1e-5)
    shift = bt - m * scale
    return scale, shift


def kernel(xyz, points, W0, b0, g0, bt0, W1, b1, g1, bt1, W2, b2, g2, bt2):
    xyz_t = jnp.transpose(xyz, (0, 2, 1))
    xyz_r = xyz_t.reshape(B, 3, NR, NC_)
    nx, ny, nz = _fps(xyz_r)
    gidx = _ballq(xyz_t, nx, ny, nz)

    table = jnp.concatenate(
        [xyz, points, jnp.zeros((B, N, TW - 9), jnp.float32)], axis=-1
    ).reshape(B * N, TW)
    gathered = _sc_gather(table, gidx.reshape(-1))

    new_xyz = jnp.stack([nx, ny, nz], axis=-1)
    nxpad = jnp.concatenate(
        [new_xyz.reshape(B * S, 3), jnp.zeros((B * S, TW - 3), jnp.float32)],
        axis=-1)

    w0p = jnp.zeros((TW, 32), jnp.float32).at[:9, :].set(W0.T)
    z1, s1, q1 = _p1(gathered, nxpad, w0p, b0)
    sc1, sh1 = _stats(s1, q1, g0, bt0)
    z2, s2, q2 = _pmid(z1, sc1, sh1, W1.T, b1)
    sc2, sh2 = _stats(s2, q2, g1, bt1)
    z3, s3, q3 = _pmid(z2, sc2, sh2, W2.T, b2)
    sc3, sh3 = _stats(s3, q3, g2, bt2)
    new_points = _p4(z3, sc3, sh3).reshape(B, S, 64)
    return (new_xyz, new_points)


# speedup vs baseline: 23.3820x; 1.7320x over previous
import functools

import jax
import jax.numpy as jnp
from jax import lax
from jax.experimental import pallas as pl
from jax.experimental.pallas import tpu as pltpu
from jax.experimental.pallas import tpu_sc as plsc

B = 8
N = 16384
S = 512
K = 32
R2 = 0.2 ** 2
NR = 8
NC_ = N // NR
CHUNK = 512
NCHUNKS = N // CHUNK
BIGF = 1e10
BIGI = 1 << 30


def _fps_body(xr_ref, nx_ref, ny_ref, nz_ref, dist_ref):
    x = xr_ref[:, 0]
    y = xr_ref[:, 1]
    z = xr_ref[:, 2]
    li = (lax.broadcasted_iota(jnp.int32, (B, NR, NC_), 1) * NC_
          + lax.broadcasted_iota(jnp.int32, (B, NR, NC_), 2))
    col = lax.broadcasted_iota(jnp.int32, (B, S), 1)
    dist_ref[...] = jnp.full((B, NR, NC_), BIGF, jnp.float32)

    def step(i, far):
        fmask = li == far
        cx = jnp.sum(jnp.where(fmask, x, 0.0), axis=(1, 2), keepdims=True)
        cy = jnp.sum(jnp.where(fmask, y, 0.0), axis=(1, 2), keepdims=True)
        cz = jnp.sum(jnp.where(fmask, z, 0.0), axis=(1, 2), keepdims=True)
        sel = col == i
        nx_ref[...] = jnp.where(sel, cx[:, :, 0], nx_ref[...])
        ny_ref[...] = jnp.where(sel, cy[:, :, 0], ny_ref[...])
        nz_ref[...] = jnp.where(sel, cz[:, :, 0], nz_ref[...])
        dx = x - cx
        dy = y - cy
        dz = z - cz
        d = (dx * dx + dy * dy) + dz * dz
        dist = jnp.minimum(dist_ref[...], d)
        dist_ref[...] = dist
        m = jnp.max(dist, axis=(1, 2), keepdims=True)
        return jnp.min(jnp.where(dist == m, li, BIGI), axis=(1, 2),
                       keepdims=True)

    lax.fori_loop(0, S, step, jnp.zeros((B, 1, 1), jnp.int32))


def _fps(xyz_r):
    out = [jax.ShapeDtypeStruct((B, S), jnp.float32)] * 3
    spec = pl.BlockSpec((B, S), lambda: (0, 0))
    return pl.pallas_call(
        _fps_body,
        in_specs=[pl.BlockSpec((B, 3, NR, NC_), lambda: (0, 0, 0, 0))],
        out_specs=[spec, spec, spec],
        out_shape=out,
        scratch_shapes=[pltpu.VMEM((B, NR, NC_), jnp.float32)],
    )(xyz_r)


def _ballq_body(xt_ref, nx_ref, ny_ref, nz_ref, out_ref):
    b = pl.program_id(0)
    sx = nx_ref[0]
    sy = ny_ref[0]
    sz = nz_ref[0]
    s2 = (sx * sx + sy * sy) + sz * sz
    src = jnp.concatenate([sx, sy, sz], axis=1)
    kio = lax.broadcasted_iota(jnp.int32, (S, K), 1)

    def cond(carry):
        ci, count, _ = carry
        return jnp.logical_and(ci < NCHUNKS, jnp.min(count) < K)

    def body(carry):
        ci, count, idxout = carry
        off = pl.multiple_of(ci * CHUNK, CHUNK)
        dx = xt_ref[0, 0:1, pl.ds(off, CHUNK)]
        dy = xt_ref[0, 1:2, pl.ds(off, CHUNK)]
        dz = xt_ref[0, 2:3, pl.ds(off, CHUNK)]
        dstc = jnp.concatenate([dx, dy, dz], axis=0)
        dot = lax.dot_general(src, dstc, (((1,), (0,)), ((), ())))
        d2 = (dx * dx + dy * dy) + dz * dz
        d = ((-2.0 * dot) + s2) + d2
        gi = lax.broadcasted_iota(jnp.int32, (S, CHUNK), 1) + off
        mi = jnp.where(d > R2, BIGI, gi)

        def ex_cond(c2):
            t, go, _, _, _ = c2
            return jnp.logical_and(t < K, go)

        def extract(c2):
            t, _, count2, idxout2, mi2 = c2
            first = jnp.min(mi2, axis=1, keepdims=True)
            take = jnp.logical_and(first < BIGI, count2 < K)
            upd = jnp.logical_and(kio == count2, take)
            idxout2 = jnp.where(upd, first, idxout2)
            mi2 = jnp.where(mi2 == first, BIGI, mi2)
            count2 = count2 + take.astype(jnp.int32)
            return t + 1, jnp.any(take), count2, idxout2, mi2

        _, _, count, idxout, _ = lax.while_loop(
            ex_cond, extract,
            (jnp.int32(0), jnp.bool_(True), count, idxout, mi))
        return ci + 1, count, idxout

    init = (jnp.int32(0),
            jnp.zeros((S, 1), jnp.int32),
            jnp.zeros((S, K), jnp.int32))
    _, count, idxout = lax.while_loop(cond, body, init)
    idxout = jnp.where(kio >= count, idxout[:, 0:1], idxout)
    out_ref[0] = idxout + b * N


def _ballq(xyz_t, nx, ny, nz):
    nspec = pl.BlockSpec((1, S, 1), lambda b: (b, 0, 0))
    return pl.pallas_call(
        _ballq_body,
        grid=(B,),
        in_specs=[pl.BlockSpec((1, 3, N), lambda b: (b, 0, 0)),
                  nspec, nspec, nspec],
        out_specs=pl.BlockSpec((1, S, K), lambda b: (b, 0, 0)),
        out_shape=jax.ShapeDtypeStruct((B, S, K), jnp.int32),
    )(xyz_t, nx[..., None], ny[..., None], nz[..., None])


ROWS = B * S * K
NW = 32
RPW = ROWS // NW
GC = 128
NCH = RPW // GC
TW = 16


def _sc_gather(table, gidx):
    mesh = plsc.VectorSubcoreMesh(core_axis_name="c", subcore_axis_name="s")

    @functools.partial(
        pl.kernel,
        mesh=mesh,
        compiler_params=pltpu.CompilerParams(use_tc_tiling_on_sc=False),
        out_type=jax.ShapeDtypeStruct((ROWS, TW), jnp.float32),
        scratch_types=[
            pltpu.VMEM((NCH, GC), jnp.int32),
            pltpu.VMEM((GC, TW), jnp.float32),
            pltpu.SemaphoreType.DMA,
        ],
    )
    def k(table_hbm, idx_hbm, out_hbm, idx_v, row_v, sem):
        wid = lax.axis_index("s") * 2 + lax.axis_index("c")
        pltpu.sync_copy(idx_hbm.at[wid], idx_v)
        base = wid * RPW

        def body(c, carry):
            pltpu.async_copy(table_hbm.at[idx_v.at[c]], row_v, sem).wait()
            pltpu.sync_copy(row_v, out_hbm.at[pl.ds(base + c * GC, GC)])
            return carry

        lax.fori_loop(0, NCH, body, jnp.int32(0))

    return k(table, gidx.reshape(NW, NCH, GC))


G = 256
RB = G * K
NBLK = (B * S) // G


def _p1_body(g_ref, nx_ref, w_ref, b_ref, z_ref, s_ref, q_ref, sa, qa):
    i = pl.program_id(0)
    xb = g_ref[...]
    nb = nx_ref[...]
    x0 = (xb - nb[:, None, :]).reshape(RB, TW)
    z = lax.dot_general(x0, w_ref[...], (((1,), (0,)), ((), ()))) + b_ref[...]
    z_ref[...] = z

    @pl.when(i == 0)
    def _():
        sa[...] = jnp.zeros_like(sa)
        qa[...] = jnp.zeros_like(qa)

    sa[...] += jnp.sum(z, axis=0, keepdims=True)
    qa[...] += jnp.sum(z * z, axis=0, keepdims=True)

    @pl.when(i == NBLK - 1)
    def _():
        s_ref[...] = sa[...]
        q_ref[...] = qa[...]


def _p1(gathered, nxpad, w0p, b0):
    co = w0p.shape[1]
    return pl.pallas_call(
        _p1_body,
        grid=(NBLK,),
        in_specs=[
            pl.BlockSpec((G, K, TW), lambda i: (i, 0, 0)),
            pl.BlockSpec((G, TW), lambda i: (i, 0)),
            pl.BlockSpec((TW, co), lambda i: (0, 0)),
            pl.BlockSpec((1, co), lambda i: (0, 0)),
        ],
        out_specs=[
            pl.BlockSpec((RB, co), lambda i: (i, 0)),
            pl.BlockSpec((1, co), lambda i: (0, 0)),
            pl.BlockSpec((1, co), lambda i: (0, 0)),
        ],
        out_shape=[
            jax.ShapeDtypeStruct((ROWS, co), jnp.float32),
            jax.ShapeDtypeStruct((1, co), jnp.float32),
            jax.ShapeDtypeStruct((1, co), jnp.float32),
        ],
        scratch_shapes=[pltpu.VMEM((1, co), jnp.float32),
                        pltpu.VMEM((1, co), jnp.float32)],
    )(gathered.reshape(B * S, K, TW), nxpad, w0p, b0[None, :])


def _pmid_body(z_ref, sc_ref, sh_ref, w_ref, b_ref, z2_ref, s_ref, q_ref,
               sa, qa):
    i = pl.program_id(0)
    y = jnp.maximum(z_ref[...] * sc_ref[...] + sh_ref[...], 0.0)
    z = lax.dot_general(y, w_ref[...], (((1,), (0,)), ((), ()))) + b_ref[...]
    z2_ref[...] = z

    @pl.when(i == 0)
    def _():
        sa[...] = jnp.zeros_like(sa)
        qa[...] = jnp.zeros_like(qa)

    sa[...] += jnp.sum(z, axis=0, keepdims=True)
    qa[...] += jnp.sum(z * z, axis=0, keepdims=True)

    @pl.when(i == NBLK - 1)
    def _():
        s_ref[...] = sa[...]
        q_ref[...] = qa[...]


def _pmid(zin, scale, shift, wp, bv):
    ci = zin.shape[1]
    co = wp.shape[1]
    return pl.pallas_call(
        _pmid_body,
        grid=(NBLK,),
        in_specs=[
            pl.BlockSpec((RB, ci), lambda i: (i, 0)),
            pl.BlockSpec((1, ci), lambda i: (0, 0)),
            pl.BlockSpec((1, ci), lambda i: (0, 0)),
            pl.BlockSpec((ci, co), lambda i: (0, 0)),
            pl.BlockSpec((1, co), lambda i: (0, 0)),
        ],
        out_specs=[
            pl.BlockSpec((RB, co), lambda i: (i, 0)),
            pl.BlockSpec((1, co), lambda i: (0, 0)),
            pl.BlockSpec((1, co), lambda i: (0, 0)),
        ],
        out_shape=[
            jax.ShapeDtypeStruct((ROWS, co), jnp.float32),
            jax.ShapeDtypeStruct((1, co), jnp.float32),
            jax.ShapeDtypeStruct((1, co), jnp.float32),
        ],
        scratch_shapes=[pltpu.VMEM((1, co), jnp.float32),
                        pltpu.VMEM((1, co), jnp.float32)],
    )(zin, scale[None, :], shift[None, :], wp, bv[None, :])


def _p4_body(z_ref, sc_ref, sh_ref, out_ref):
    y = jnp.maximum(z_ref[...] * sc_ref[...][None] + sh_ref[...][None], 0.0)
    out_ref[...] = jnp.max(y, axis=1)


def _p4(z3, scale, shift):
    co = z3.shape[1]
    return pl.pallas_call(
        _p4_body,
        grid=(NBLK,),
        in_specs=[
            pl.BlockSpec((G, K, co), lambda i: (i, 0, 0)),
            pl.BlockSpec((1, co), lambda i: (0, 0)),
            pl.BlockSpec((1, co), lambda i: (0, 0)),
        ],
        out_specs=pl.BlockSpec((G, co), lambda i: (i, 0)),
        out_shape=jax.ShapeDtypeStruct((B * S, co), jnp.float32),
    )(z3.reshape(B * S, K, co), scale[None, :], shift[None, :])


def _stats(s, q, g, bt):
    m = s[0] / ROWS
    v = q[0] / ROWS - m * m
    scale = g * lax.rsqrt(v + 1e-5)
    shift = bt - m * scale
    return scale, shift


def kernel(xyz, points, W0, b0, g0, bt0, W1, b1, g1, bt1, W2, b2, g2, bt2):
    xyz_t = jnp.transpose(xyz, (0, 2, 1))
    xyz_r = xyz_t.reshape(B, 3, NR, NC_)
    nx, ny, nz = _fps(xyz_r)
    gidx = _ballq(xyz_t, nx, ny, nz)

    table = jnp.concatenate(
        [xyz, points, jnp.zeros((B, N, TW - 9), jnp.float32)], axis=-1
    ).reshape(B * N, TW)
    gathered = _sc_gather(table, gidx.reshape(-1))

    new_xyz = jnp.stack([nx, ny, nz], axis=-1)
    nxpad = jnp.concatenate(
        [new_xyz.reshape(B * S, 3), jnp.zeros((B * S, TW - 3), jnp.float32)],
        axis=-1)

    w0p = jnp.zeros((TW, 32), jnp.float32).at[:9, :].set(W0.T)
    z1, s1, q1 = _p1(gathered, nxpad, w0p, b0)
    sc1, sh1 = _stats(s1, q1, g0, bt0)
    z2, s2, q2 = _pmid(z1, sc1, sh1, W1.T, b1)
    sc2, sh2 = _stats(s2, q2, g1, bt1)
    z3, s3, q3 = _pmid(z2, sc2, sh2, W2.T, b2)
    sc3, sh3 = _stats(s3, q3, g2, bt2)
    new_points = _p4(z3, sc3, sh3).reshape(B, S, 64)
    return (new_xyz, new_points)
